# baseline (speedup 1.0000x reference)
import functools

import jax
import jax.numpy as jnp
from jax import lax
from jax.experimental import pallas as pl
from jax.experimental.pallas import tpu as pltpu
from jax.experimental.pallas import tpu_sc as plsc

N = 10000
D = 128
NG = 64
NC = 2
NS = 16
NW = NC * NS
CHUNK = 128
TOT_CHUNKS = 2560
EP = TOT_CHUNKS * CHUNK
CPT = TOT_CHUNKS // NS
C0 = 144
C1 = CPT - C0
DEG_CPW = TOT_CHUNKS // NW
NP = 10112
RPT = NP // NS
SINK = N
DEGW = 16
IDXB = 8
DEG_NB = 8
R = 1000


@functools.cache
def _sc_mesh():
    return plsc.VectorSubcoreMesh(
        core_axis_name="c", subcore_axis_name="s",
        num_cores=NC, num_subcores=NS)


@functools.cache
def _sc_mesh1():
    return plsc.VectorSubcoreMesh(
        core_axis_name="c", subcore_axis_name="s",
        num_cores=1, num_subcores=NS)


def _zero_fill(ref, nrows, ncols):
    def body(i, carry):
        for j in range(ncols // 16):
            ref[i, pl.ds(j * 16, 16)] = jnp.zeros((16,), jnp.float32)
        return carry
    lax.fori_loop(0, nrows, body, 0)


def _sc_deg_body(eidx_hbm, out_hbm, acc_sh, idx_v, ones_v, zero_v, sem):
    cid = lax.axis_index("c")
    sid = lax.axis_index("s")

    def fill_ones(i, carry):
        ones_v[i, :] = jnp.ones((16,), jnp.float32)
        return carry
    lax.fori_loop(0, CHUNK, fill_ones, 0)
    _zero_fill(zero_v, RPT, DEGW)
    pltpu.sync_copy(zero_v, acc_sh.at[pl.ds(sid * RPT, RPT)])
    wid = cid * NS + sid
    pltpu.sync_copy(eidx_hbm.at[pl.ds(wid * DEG_CPW, DEG_CPW)], idx_v)
    plsc.subcore_barrier()

    def group(g, carry):
        descs = []
        for b in range(DEG_NB):
            c = g * DEG_NB + b
            descs.append(pltpu.async_copy(
                ones_v, acc_sh.at[idx_v.at[c, 1]], sem, add=True))
        for d in descs:
            d.wait()
        return carry
    lax.fori_loop(0, DEG_CPW // DEG_NB, group, 0)

    plsc.subcore_barrier()
    pltpu.sync_copy(acc_sh.at[pl.ds(sid * RPT, RPT)],
                    out_hbm.at[cid, pl.ds(sid * RPT, RPT)])


@functools.cache
def _deg_call():
    return pl.kernel(
        _sc_deg_body,
        out_type=jax.ShapeDtypeStruct((NC, NP, DEGW), jnp.float32),
        mesh=_sc_mesh(),
        scratch_types=[
            pltpu.VMEM_SHARED((NP, DEGW), jnp.float32),
            pltpu.VMEM((DEG_CPW, 2, CHUNK), jnp.int32),
            pltpu.VMEM((CHUNK, DEGW), jnp.float32),
            pltpu.VMEM((RPT, DEGW), jnp.float32),
            pltpu.SemaphoreType.DMA,
        ],
    )


def _agg_pipeline(tbl_hbm, eidx_hbm, acc_sh, idx_v, rows,
                  sem_i, sem_g, sem_s, start, n):
    pltpu.async_copy(eidx_hbm.at[pl.ds(start, IDXB)], idx_v.at[0],
                     sem_i).wait()
    pltpu.async_copy(eidx_hbm.at[pl.ds(start + IDXB, IDXB)], idx_v.at[1],
                     sem_i)
    pltpu.async_copy(tbl_hbm.at[idx_v.at[0, 0, 0]], rows.at[0], sem_g[0])

    def step2(g, carry):
        for par in range(2):
            c = 2 * g + par
            b = par
            pltpu.make_async_copy(tbl_hbm.at[pl.ds(0, CHUNK)],
                                  rows.at[b], sem_g[b]).wait()
            blk = (c // IDXB) % 3
            roff = c % IDXB
            pltpu.async_copy(rows.at[b], acc_sh.at[idx_v.at[blk, roff, 1]],
                             sem_s[b], add=True)
            c1 = c + 1

            @pl.when(c1 < n)
            def _():
                @pl.when(c1 % IDXB == 0)
                def _():
                    pltpu.make_async_copy(
                        eidx_hbm.at[pl.ds(0, IDXB)],
                        idx_v.at[0], sem_i).wait()

                    @pl.when(c1 + IDXB < n)
                    def _():
                        nb = c1 // IDXB + 1
                        pltpu.async_copy(
                            eidx_hbm.at[pl.ds(start + IDXB * nb, IDXB)],
                            idx_v.at[nb % 3], sem_i)

                @pl.when(c >= 1)
                def _():
                    pltpu.make_async_copy(tbl_hbm.at[pl.ds(0, CHUNK)],
                                          rows.at[1 - b], sem_s[1 - b]).wait()
                blk1 = (c1 // IDXB) % 3
                pltpu.async_copy(tbl_hbm.at[idx_v.at[blk1, c1 % IDXB, 0]],
                                 rows.at[1 - b], sem_g[1 - b])
        return carry
    lax.fori_loop(0, n // 2, step2, 0)

    pltpu.make_async_copy(tbl_hbm.at[pl.ds(0, CHUNK)], rows.at[0],
                          sem_s[0]).wait()
    pltpu.make_async_copy(tbl_hbm.at[pl.ds(0, CHUNK)], rows.at[1],
                          sem_s[1]).wait()


def _sc_agg_body(tbl_hbm, eidx_hbm, out_hbm,
                 acc_sh, idx_v, rows, sem_i, sem_g0, sem_g1, sem_s0, sem_s1):
    cid = lax.axis_index("c")
    sid = lax.axis_index("s")
    sem_g = (sem_g0, sem_g1)
    sem_s = (sem_s0, sem_s1)

    _zero_fill(rows.at[0], CHUNK, D)
    for k in range(RPT // CHUNK):
        pltpu.sync_copy(rows.at[0],
                        acc_sh.at[pl.ds(sid * RPT + k * CHUNK, CHUNK)])
    rem = RPT % CHUNK
    pltpu.sync_copy(rows.at[0, pl.ds(0, rem)],
                    acc_sh.at[pl.ds(sid * RPT + (RPT // CHUNK) * CHUNK, rem)])
    plsc.subcore_barrier()

    _agg_pipeline(tbl_hbm, eidx_hbm, acc_sh, idx_v, rows,
                  sem_i, sem_g, sem_s, sid * CPT, CPT)

    plsc.subcore_barrier()
    pltpu.sync_copy(acc_sh.at[pl.ds(sid * RPT, RPT)],
                    out_hbm.at[0, pl.ds(sid * RPT, RPT)])


@functools.cache
def _agg_call():
    return pl.kernel(
        _sc_agg_body,
        out_type=jax.ShapeDtypeStruct((1, NP, D), jnp.float32),
        mesh=_sc_mesh1(),
        scratch_types=[
            pltpu.VMEM_SHARED((NP, D), jnp.float32),
            pltpu.VMEM((3, IDXB, 2, CHUNK), jnp.int32),
            pltpu.VMEM((2, CHUNK, D), jnp.float32),
            pltpu.SemaphoreType.DMA,
            pltpu.SemaphoreType.DMA,
            pltpu.SemaphoreType.DMA,
            pltpu.SemaphoreType.DMA,
            pltpu.SemaphoreType.DMA,
        ],
    )


def _tc_prep_body(degp_ref, x_ref, w_ref, dinv_ref, ht_ref):
    deg = degp_ref[0, :, 0:1] + degp_ref[1, :, 0:1] + 1.0
    dinv = lax.rsqrt(deg)
    z = lax.dot_general(x_ref[...], w_ref[...], (((1,), (1,)), ((), ())),
                        preferred_element_type=jnp.float32)
    dinv_ref[...] = dinv
    ht_ref[...] = dinv * z


_tc_prep = pl.pallas_call(
    _tc_prep_body,
    grid=(N // R,),
    in_specs=[
        pl.BlockSpec((NC, R, DEGW), lambda i: (0, i, 0)),
        pl.BlockSpec((R, D), lambda i: (i, 0)),
        pl.BlockSpec((D, D), lambda i: (0, 0)),
    ],
    out_specs=[
        pl.BlockSpec((R, 1), lambda i: (i, 0)),
        pl.BlockSpec((R, D), lambda i: (i, 0)),
    ],
    out_shape=[
        jax.ShapeDtypeStruct((N, 1), jnp.float32),
        jax.ShapeDtypeStruct((N, D), jnp.float32),
    ],
)


def _tc_layer_body(aggp_ref, htp_ref, dinv_ref, b_ref, w_ref, htn_ref):
    s = aggp_ref[0] + htp_ref[...]
    h = jnp.maximum(dinv_ref[...] * s + b_ref[...], 0.0)
    z = lax.dot_general(h, w_ref[...], (((1,), (1,)), ((), ())),
                        preferred_element_type=jnp.float32)
    htn_ref[...] = dinv_ref[...] * z


_tc_layer = pl.pallas_call(
    _tc_layer_body,
    grid=(N // R,),
    in_specs=[
        pl.BlockSpec((1, R, D), lambda i: (0, i, 0)),
        pl.BlockSpec((R, D), lambda i: (i, 0)),
        pl.BlockSpec((R, 1), lambda i: (i, 0)),
        pl.BlockSpec((1, D), lambda i: (0, 0)),
        pl.BlockSpec((D, D), lambda i: (0, 0)),
    ],
    out_specs=pl.BlockSpec((R, D), lambda i: (i, 0)),
    out_shape=jax.ShapeDtypeStruct((N, D), jnp.float32),
)


def _tc_final_body(aggp_ref, htp_ref, dinv_ref, b_ref, batch_ref,
                   hw_ref, hb_ref, out_ref, pooled_scr):
    i = pl.program_id(0)

    @pl.when(i == 0)
    def _():
        pooled_scr[...] = jnp.zeros((NG, D), jnp.float32)

    s = aggp_ref[0] + htp_ref[...]
    h = jnp.maximum(dinv_ref[...] * s + b_ref[...], 0.0)
    seg = lax.broadcasted_iota(jnp.int32, (R, NG), 1)
    onehot = (batch_ref[...] == seg).astype(jnp.float32)
    pooled_scr[...] += lax.dot_general(
        onehot, h, (((0,), (0,)), ((), ())), preferred_element_type=jnp.float32)

    @pl.when(i == N // R - 1)
    def _():
        out_ref[...] = jnp.sum(pooled_scr[...] * hw_ref[...],
                               axis=1, keepdims=True) + hb_ref[0, 0]


_tc_final = pl.pallas_call(
    _tc_final_body,
    grid=(N // R,),
    in_specs=[
        pl.BlockSpec((1, R, D), lambda i: (0, i, 0)),
        pl.BlockSpec((R, D), lambda i: (i, 0)),
        pl.BlockSpec((R, 1), lambda i: (i, 0)),
        pl.BlockSpec((1, D), lambda i: (0, 0)),
        pl.BlockSpec((R, 1), lambda i: (i, 0)),
        pl.BlockSpec((1, D), lambda i: (0, 0)),
        pl.BlockSpec((1, 1), lambda i: (0, 0)),
    ],
    out_specs=pl.BlockSpec((NG, 1), lambda i: (0, 0)),
    out_shape=jax.ShapeDtypeStruct((NG, 1), jnp.float32),
    scratch_shapes=[pltpu.VMEM((NG, D), jnp.float32)],
)


def kernel(x, edge_index, batch, W1, b1, W2, b2, W3, b3, head_w, head_b):
    x = x.astype(jnp.float32)
    src = edge_index[0]
    dst = edge_index[1]
    pad = EP - src.shape[0]
    src_p = jnp.concatenate([src, jnp.zeros((pad,), jnp.int32)])
    dst_p = jnp.concatenate([dst, jnp.full((pad,), SINK, jnp.int32)])
    eidx = jnp.stack([src_p.reshape(TOT_CHUNKS, CHUNK),
                      dst_p.reshape(TOT_CHUNKS, CHUNK)], axis=1)

    degp = _deg_call()(eidx)
    dinv, ht1 = _tc_prep(degp, x, W1)
    agg1 = _agg_call()(ht1, eidx)
    ht2 = _tc_layer(agg1, ht1, dinv, b1.reshape(1, D), W2)
    agg2 = _agg_call()(ht2, eidx)
    ht3 = _tc_layer(agg2, ht2, dinv, b2.reshape(1, D), W3)
    agg3 = _agg_call()(ht3, eidx)
    out = _tc_final(agg3, ht3, dinv, b3.reshape(1, D),
                    batch.reshape(N, 1), head_w, head_b.reshape(1, 1))
    return out.reshape(-1)


# speedup vs baseline: 1.4874x; 1.4874x over previous
import functools

import jax
import jax.numpy as jnp
from jax import lax
from jax.experimental import pallas as pl
from jax.experimental.pallas import tpu as pltpu
from jax.experimental.pallas import tpu_sc as plsc

N = 10000
D = 128
NG = 64
NC = 2
NS = 16
NW = NC * NS
CHUNK = 128
TOT_CHUNKS = 2560
EP = TOT_CHUNKS * CHUNK
CPT = TOT_CHUNKS // NS
C0 = 144
C1 = CPT - C0
DEG_CPW = TOT_CHUNKS // NW
NP = 10112
RPT = NP // NS
SINK = N
DEGW = 16
IDXB = 8
DEG_NB = 8
R = 1000


@functools.cache
def _sc_mesh():
    return plsc.VectorSubcoreMesh(
        core_axis_name="c", subcore_axis_name="s",
        num_cores=NC, num_subcores=NS)


def _zero_fill(ref, nrows, ncols):
    def body(i, carry):
        for j in range(ncols // 16):
            ref[i, pl.ds(j * 16, 16)] = jnp.zeros((16,), jnp.float32)
        return carry
    lax.fori_loop(0, nrows, body, 0)


def _sc_deg_body(eidx_hbm, out_hbm, acc_sh, idx_v, ones_v, zero_v, sem):
    cid = lax.axis_index("c")
    sid = lax.axis_index("s")

    def fill_ones(i, carry):
        ones_v[i, :] = jnp.ones((16,), jnp.float32)
        return carry
    lax.fori_loop(0, CHUNK, fill_ones, 0)
    _zero_fill(zero_v, RPT, DEGW)
    pltpu.sync_copy(zero_v, acc_sh.at[pl.ds(sid * RPT, RPT)])
    wid = cid * NS + sid
    pltpu.sync_copy(eidx_hbm.at[pl.ds(wid * DEG_CPW, DEG_CPW)], idx_v)
    plsc.subcore_barrier()

    def group(g, carry):
        descs = []
        for b in range(DEG_NB):
            c = g * DEG_NB + b
            descs.append(pltpu.async_copy(
                ones_v, acc_sh.at[idx_v.at[c, 1]], sem, add=True))
        for d in descs:
            d.wait()
        return carry
    lax.fori_loop(0, DEG_CPW // DEG_NB, group, 0)

    plsc.subcore_barrier()
    pltpu.sync_copy(acc_sh.at[pl.ds(sid * RPT, RPT)],
                    out_hbm.at[cid, pl.ds(sid * RPT, RPT)])


@functools.cache
def _deg_call():
    return pl.kernel(
        _sc_deg_body,
        out_type=jax.ShapeDtypeStruct((NC, NP, DEGW), jnp.float32),
        mesh=_sc_mesh(),
        scratch_types=[
            pltpu.VMEM_SHARED((NP, DEGW), jnp.float32),
            pltpu.VMEM((DEG_CPW, 2, CHUNK), jnp.int32),
            pltpu.VMEM((CHUNK, DEGW), jnp.float32),
            pltpu.VMEM((RPT, DEGW), jnp.float32),
            pltpu.SemaphoreType.DMA,
        ],
    )


def _agg_pipeline(tbl_hbm, eidx_hbm, acc_sh, idx_v, rows,
                  sem_i, sem_g, sem_s, start, n):
    pltpu.async_copy(eidx_hbm.at[pl.ds(start, IDXB)], idx_v.at[0],
                     sem_i).wait()
    pltpu.async_copy(eidx_hbm.at[pl.ds(start + IDXB, IDXB)], idx_v.at[1],
                     sem_i)
    pltpu.async_copy(tbl_hbm.at[idx_v.at[0, 0, 0]], rows.at[0], sem_g[0])

    def step2(g, carry):
        for par in range(2):
            c = 2 * g + par
            b = par
            pltpu.make_async_copy(tbl_hbm.at[pl.ds(0, CHUNK)],
                                  rows.at[b], sem_g[b]).wait()
            blk = (c // IDXB) % 3
            roff = c % IDXB
            pltpu.async_copy(rows.at[b], acc_sh.at[idx_v.at[blk, roff, 1]],
                             sem_s[b], add=True)
            c1 = c + 1

            @pl.when(c1 < n)
            def _():
                @pl.when(c1 % IDXB == 0)
                def _():
                    pltpu.make_async_copy(
                        eidx_hbm.at[pl.ds(0, IDXB)],
                        idx_v.at[0], sem_i).wait()

                    @pl.when(c1 + IDXB < n)
                    def _():
                        nb = c1 // IDXB + 1
                        pltpu.async_copy(
                            eidx_hbm.at[pl.ds(start + IDXB * nb, IDXB)],
                            idx_v.at[nb % 3], sem_i)

                @pl.when(c >= 1)
                def _():
                    pltpu.make_async_copy(tbl_hbm.at[pl.ds(0, CHUNK)],
                                          rows.at[1 - b], sem_s[1 - b]).wait()
                blk1 = (c1 // IDXB) % 3
                pltpu.async_copy(tbl_hbm.at[idx_v.at[blk1, c1 % IDXB, 0]],
                                 rows.at[1 - b], sem_g[1 - b])
        return carry
    lax.fori_loop(0, n // 2, step2, 0)

    pltpu.make_async_copy(tbl_hbm.at[pl.ds(0, CHUNK)], rows.at[0],
                          sem_s[0]).wait()
    pltpu.make_async_copy(tbl_hbm.at[pl.ds(0, CHUNK)], rows.at[1],
                          sem_s[1]).wait()


def _sc_agg_body(tbl_hbm, eidx_hbm, out_hbm,
                 acc_sh, idx_v, rows, sem_i, sem_g0, sem_g1, sem_s0, sem_s1):
    cid = lax.axis_index("c")
    sid = lax.axis_index("s")
    sem_g = (sem_g0, sem_g1)
    sem_s = (sem_s0, sem_s1)

    _zero_fill(rows.at[0], CHUNK, D)
    for k in range(RPT // CHUNK):
        pltpu.sync_copy(rows.at[0],
                        acc_sh.at[pl.ds(sid * RPT + k * CHUNK, CHUNK)])
    rem = RPT % CHUNK
    pltpu.sync_copy(rows.at[0, pl.ds(0, rem)],
                    acc_sh.at[pl.ds(sid * RPT + (RPT // CHUNK) * CHUNK, rem)])
    plsc.subcore_barrier()

    nch = jnp.where(cid == 0, C0, C1)
    start = jnp.where(cid == 0, sid * C0, NS * C0 + sid * C1)
    _agg_pipeline(tbl_hbm, eidx_hbm, acc_sh, idx_v, rows,
                  sem_i, sem_g, sem_s, start, nch)

    plsc.subcore_barrier()
    pltpu.sync_copy(acc_sh.at[pl.ds(sid * RPT, RPT)],
                    out_hbm.at[cid, pl.ds(sid * RPT, RPT)])


@functools.cache
def _agg_call():
    return pl.kernel(
        _sc_agg_body,
        out_type=jax.ShapeDtypeStruct((NC, NP, D), jnp.float32),
        mesh=_sc_mesh(),
        scratch_types=[
            pltpu.VMEM_SHARED((NP, D), jnp.float32),
            pltpu.VMEM((3, IDXB, 2, CHUNK), jnp.int32),
            pltpu.VMEM((2, CHUNK, D), jnp.float32),
            pltpu.SemaphoreType.DMA,
            pltpu.SemaphoreType.DMA,
            pltpu.SemaphoreType.DMA,
            pltpu.SemaphoreType.DMA,
            pltpu.SemaphoreType.DMA,
        ],
    )


def _tc_prep_body(degp_ref, x_ref, w_ref, dinv_ref, ht_ref):
    deg = degp_ref[0, :, 0:1] + degp_ref[1, :, 0:1] + 1.0
    dinv = lax.rsqrt(deg)
    z = lax.dot_general(x_ref[...], w_ref[...], (((1,), (1,)), ((), ())),
                        preferred_element_type=jnp.float32)
    dinv_ref[...] = dinv
    ht_ref[...] = dinv * z


_tc_prep = pl.pallas_call(
    _tc_prep_body,
    grid=(N // R,),
    in_specs=[
        pl.BlockSpec((NC, R, DEGW), lambda i: (0, i, 0)),
        pl.BlockSpec((R, D), lambda i: (i, 0)),
        pl.BlockSpec((D, D), lambda i: (0, 0)),
    ],
    out_specs=[
        pl.BlockSpec((R, 1), lambda i: (i, 0)),
        pl.BlockSpec((R, D), lambda i: (i, 0)),
    ],
    out_shape=[
        jax.ShapeDtypeStruct((N, 1), jnp.float32),
        jax.ShapeDtypeStruct((N, D), jnp.float32),
    ],
)


def _tc_layer_body(aggp_ref, htp_ref, dinv_ref, b_ref, w_ref, htn_ref):
    s = aggp_ref[0] + aggp_ref[1] + htp_ref[...]
    h = jnp.maximum(dinv_ref[...] * s + b_ref[...], 0.0)
    z = lax.dot_general(h, w_ref[...], (((1,), (1,)), ((), ())),
                        preferred_element_type=jnp.float32)
    htn_ref[...] = dinv_ref[...] * z


_tc_layer = pl.pallas_call(
    _tc_layer_body,
    grid=(N // R,),
    in_specs=[
        pl.BlockSpec((NC, R, D), lambda i: (0, i, 0)),
        pl.BlockSpec((R, D), lambda i: (i, 0)),
        pl.BlockSpec((R, 1), lambda i: (i, 0)),
        pl.BlockSpec((1, D), lambda i: (0, 0)),
        pl.BlockSpec((D, D), lambda i: (0, 0)),
    ],
    out_specs=pl.BlockSpec((R, D), lambda i: (i, 0)),
    out_shape=jax.ShapeDtypeStruct((N, D), jnp.float32),
)


def _tc_final_body(aggp_ref, htp_ref, dinv_ref, b_ref, batch_ref,
                   hw_ref, hb_ref, out_ref, pooled_scr):
    i = pl.program_id(0)

    @pl.when(i == 0)
    def _():
        pooled_scr[...] = jnp.zeros((NG, D), jnp.float32)

    s = aggp_ref[0] + aggp_ref[1] + htp_ref[...]
    h = jnp.maximum(dinv_ref[...] * s + b_ref[...], 0.0)
    seg = lax.broadcasted_iota(jnp.int32, (R, NG), 1)
    onehot = (batch_ref[...] == seg).astype(jnp.float32)
    pooled_scr[...] += lax.dot_general(
        onehot, h, (((0,), (0,)), ((), ())), preferred_element_type=jnp.float32)

    @pl.when(i == N // R - 1)
    def _():
        out_ref[...] = jnp.sum(pooled_scr[...] * hw_ref[...],
                               axis=1, keepdims=True) + hb_ref[0, 0]


_tc_final = pl.pallas_call(
    _tc_final_body,
    grid=(N // R,),
    in_specs=[
        pl.BlockSpec((NC, R, D), lambda i: (0, i, 0)),
        pl.BlockSpec((R, D), lambda i: (i, 0)),
        pl.BlockSpec((R, 1), lambda i: (i, 0)),
        pl.BlockSpec((1, D), lambda i: (0, 0)),
        pl.BlockSpec((R, 1), lambda i: (i, 0)),
        pl.BlockSpec((1, D), lambda i: (0, 0)),
        pl.BlockSpec((1, 1), lambda i: (0, 0)),
    ],
    out_specs=pl.BlockSpec((NG, 1), lambda i: (0, 0)),
    out_shape=jax.ShapeDtypeStruct((NG, 1), jnp.float32),
    scratch_shapes=[pltpu.VMEM((NG, D), jnp.float32)],
)


def kernel(x, edge_index, batch, W1, b1, W2, b2, W3, b3, head_w, head_b):
    x = x.astype(jnp.float32)
    src = edge_index[0]
    dst = edge_index[1]
    pad = EP - src.shape[0]
    src_p = jnp.concatenate([src, jnp.zeros((pad,), jnp.int32)])
    dst_p = jnp.concatenate([dst, jnp.full((pad,), SINK, jnp.int32)])
    eidx = jnp.stack([src_p.reshape(TOT_CHUNKS, CHUNK),
                      dst_p.reshape(TOT_CHUNKS, CHUNK)], axis=1)

    degp = _deg_call()(eidx)
    dinv, ht1 = _tc_prep(degp, x, W1)
    agg1 = _agg_call()(ht1, eidx)
    ht2 = _tc_layer(agg1, ht1, dinv, b1.reshape(1, D), W2)
    agg2 = _agg_call()(ht2, eidx)
    ht3 = _tc_layer(agg2, ht2, dinv, b2.reshape(1, D), W3)
    agg3 = _agg_call()(ht3, eidx)
    out = _tc_final(agg3, ht3, dinv, b3.reshape(1, D),
                    batch.reshape(N, 1), head_w, head_b.reshape(1, 1))
    return out.reshape(-1)
